# baseline (speedup 1.0000x reference)
import functools

import jax
import jax.numpy as jnp
from jax.experimental import pallas as pl
from jax.experimental.pallas import tpu as pltpu

NUM_EXPERTS = 64
TOP_K = 8
HIDDEN = 1024
INTER = 512
E_BLK = 4


def _moe_body(hs_ref, gw_ref, gp_ref, up_ref, dp_ref, out_ref, logits_ref,
              comb_ref):
    i = pl.program_id(0)
    hs = hs_ref[...]
    T = hs.shape[0]

    @pl.when(i == 0)
    def _router():
        logits = jax.lax.dot_general(
            hs, gw_ref[...], (((1,), (1,)), ((), ())),
            preferred_element_type=jnp.float32)
        logits_ref[...] = logits
        probs = jax.nn.softmax(logits, axis=1)
        E = probs.shape[1]
        colid = jax.lax.broadcasted_iota(jnp.int32, (T, E), 1)
        comb = jnp.zeros_like(probs)
        p = probs
        for _ in range(TOP_K):
            m = jnp.max(p, axis=1, keepdims=True)
            idx = jnp.where(p == m, colid, E)
            sel = colid == jnp.min(idx, axis=1, keepdims=True)
            comb = jnp.where(sel, p, comb)
            p = jnp.where(sel, -1.0, p)
        comb = comb / jnp.sum(comb, axis=1, keepdims=True)
        comb_ref[...] = comb
        out_ref[...] = jnp.zeros_like(out_ref)

    I = gp_ref.shape[1]
    H = hs.shape[1]
    gp = gp_ref[...].reshape(E_BLK * I, H)
    up = up_ref[...].reshape(E_BLK * I, H)
    g = jax.lax.dot_general(hs, gp, (((1,), (1,)), ((), ())),
                            preferred_element_type=jnp.float32)
    u = jax.lax.dot_general(hs, up, (((1,), (1,)), ((), ())),
                            preferred_element_type=jnp.float32)
    a = g * jax.nn.sigmoid(g) * u

    E = comb_ref.shape[1]
    colid = jax.lax.broadcasted_iota(jnp.int32, (T, E), 1)
    comb = comb_ref[...]
    acc = out_ref[...]
    for j in range(E_BLK):
        e = i * E_BLK + j
        w = jnp.sum(jnp.where(colid == e, comb, 0.0), axis=1,
                    keepdims=True)
        aj = (a[:, j * I:(j + 1) * I]) * w
        acc = acc + jax.lax.dot_general(
            aj, dp_ref[j], (((1,), (1,)), ((), ())),
            preferred_element_type=jnp.float32)
    out_ref[...] = acc


@functools.partial(jax.jit, static_argnames=())
def kernel(hidden_states, gate_w, gate_proj, up_proj, down_proj):
    B, S, H = hidden_states.shape
    T = B * S
    hs = hidden_states.reshape(T, H)
    E = gate_w.shape[0]
    I = gate_proj.shape[1]

    final, logits = pl.pallas_call(
        _moe_body,
        grid=(E // E_BLK,),
        in_specs=[
            pl.BlockSpec((T, H), lambda i: (0, 0)),
            pl.BlockSpec((E, H), lambda i: (0, 0)),
            pl.BlockSpec((E_BLK, I, H), lambda i: (i, 0, 0)),
            pl.BlockSpec((E_BLK, I, H), lambda i: (i, 0, 0)),
            pl.BlockSpec((E_BLK, H, I), lambda i: (i, 0, 0)),
        ],
        out_specs=[
            pl.BlockSpec((T, H), lambda i: (0, 0)),
            pl.BlockSpec((T, E), lambda i: (0, 0)),
        ],
        out_shape=[
            jax.ShapeDtypeStruct((T, H), jnp.float32),
            jax.ShapeDtypeStruct((T, E), jnp.float32),
        ],
        scratch_shapes=[pltpu.VMEM((T, E), jnp.float32)],
        compiler_params=pltpu.CompilerParams(
            dimension_semantics=("arbitrary",),
        ),
    )(hs, gate_w, gate_proj, up_proj, down_proj)

    return final.reshape(B, S, H), logits


# speedup vs baseline: 1.0454x; 1.0454x over previous
import functools

import jax
import jax.numpy as jnp
from jax.experimental import pallas as pl
from jax.experimental.pallas import tpu as pltpu

NUM_EXPERTS = 64
TOP_K = 8
E_BLK = 2
SPLIT = 4


def _moe_body(*refs):
    hs_ref, gw_ref = refs[0], refs[1]
    gp_refs = refs[2:2 + SPLIT]
    up_refs = refs[2 + SPLIT:2 + 2 * SPLIT]
    dp_refs = refs[2 + 2 * SPLIT:2 + 3 * SPLIT]
    out_ref, logits_ref, comb_ref = refs[2 + 3 * SPLIT:]

    i = pl.program_id(0)
    hs = hs_ref[...]
    T, H = hs.shape
    Ip = gp_refs[0].shape[1]
    I = Ip * SPLIT
    Hp = dp_refs[0].shape[1]

    @pl.when(i == 0)
    def _router():
        logits = jax.lax.dot_general(
            hs, gw_ref[...], (((1,), (1,)), ((), ())),
            preferred_element_type=jnp.float32)
        logits_ref[...] = logits
        probs = jax.nn.softmax(logits, axis=1)
        E = probs.shape[1]
        colid = jax.lax.broadcasted_iota(jnp.int32, (T, E), 1)
        comb = jnp.zeros_like(probs)
        p = probs
        for _ in range(TOP_K):
            m = jnp.max(p, axis=1, keepdims=True)
            idx = jnp.where(p == m, colid, E)
            sel = colid == jnp.min(idx, axis=1, keepdims=True)
            comb = jnp.where(sel, p, comb)
            p = jnp.where(sel, -1.0, p)
        comb = comb / jnp.sum(comb, axis=1, keepdims=True)
        comb_ref[...] = comb
        out_ref[...] = jnp.zeros_like(out_ref)

    a_parts = [None] * (E_BLK * SPLIT)
    for p in range(SPLIT):
        gp = gp_refs[p][...].reshape(E_BLK * Ip, H)
        up = up_refs[p][...].reshape(E_BLK * Ip, H)
        g = jax.lax.dot_general(hs, gp, (((1,), (1,)), ((), ())),
                                preferred_element_type=jnp.float32)
        u = jax.lax.dot_general(hs, up, (((1,), (1,)), ((), ())),
                                preferred_element_type=jnp.float32)
        a = g * jax.nn.sigmoid(g) * u
        for j in range(E_BLK):
            a_parts[j * SPLIT + p] = a[:, j * Ip:(j + 1) * Ip]

    E = comb_ref.shape[1]
    colid = jax.lax.broadcasted_iota(jnp.int32, (T, E), 1)
    comb = comb_ref[...]
    aw = []
    for j in range(E_BLK):
        e = i * E_BLK + j
        w = jnp.sum(jnp.where(colid == e, comb, 0.0), axis=1,
                    keepdims=True)
        aj = jnp.concatenate(a_parts[j * SPLIT:(j + 1) * SPLIT], axis=1)
        aw.append(aj * w)

    for p in range(SPLIT):
        acc = out_ref[:, p * Hp:(p + 1) * Hp]
        for j in range(E_BLK):
            acc = acc + jax.lax.dot_general(
                aw[j], dp_refs[p][j], (((1,), (1,)), ((), ())),
                preferred_element_type=jnp.float32)
        out_ref[:, p * Hp:(p + 1) * Hp] = acc


@functools.partial(jax.jit, static_argnames=())
def kernel(hidden_states, gate_w, gate_proj, up_proj, down_proj):
    B, S, H = hidden_states.shape
    T = B * S
    hs = hidden_states.reshape(T, H)
    E = gate_w.shape[0]
    I = gate_proj.shape[1]
    Ip = I // SPLIT
    Hp = H // SPLIT

    in_specs = [
        pl.BlockSpec((T, H), lambda i: (0, 0)),
        pl.BlockSpec((E, H), lambda i: (0, 0)),
    ]
    for p in range(SPLIT):
        in_specs.append(
            pl.BlockSpec((E_BLK, Ip, H), lambda i, p=p: (i, p, 0)))
    for p in range(SPLIT):
        in_specs.append(
            pl.BlockSpec((E_BLK, Ip, H), lambda i, p=p: (i, p, 0)))
    for p in range(SPLIT):
        in_specs.append(
            pl.BlockSpec((E_BLK, Hp, I), lambda i, p=p: (i, p, 0)))

    final, logits = pl.pallas_call(
        _moe_body,
        grid=(E // E_BLK,),
        in_specs=in_specs,
        out_specs=[
            pl.BlockSpec((T, H), lambda i: (0, 0)),
            pl.BlockSpec((T, E), lambda i: (0, 0)),
        ],
        out_shape=[
            jax.ShapeDtypeStruct((T, H), jnp.float32),
            jax.ShapeDtypeStruct((T, E), jnp.float32),
        ],
        scratch_shapes=[pltpu.VMEM((T, E), jnp.float32)],
        compiler_params=pltpu.CompilerParams(
            dimension_semantics=("arbitrary",),
        ),
    )(hs, gate_w,
      *([gate_proj] * SPLIT), *([up_proj] * SPLIT), *([down_proj] * SPLIT))

    return final.reshape(B, S, H), logits
